# baseline (speedup 1.0000x reference)
import jax
import jax.numpy as jnp
from jax import lax
from jax.experimental import pallas as pl

NUM_CLASSES = 1000
BATCH = 16384
ROWS = 512


def _onehot_block(x_ref, out_ref):
    x = x_ref[...]
    cols = lax.broadcasted_iota(jnp.int32, (ROWS, NUM_CLASSES), 1)
    out_ref[...] = jnp.where(x == cols, 1.0, 0.0).astype(jnp.float32)


def kernel(x):
    x = x.astype(jnp.int32)
    grid = BATCH // ROWS
    x2 = x.reshape(BATCH, 1)
    return pl.pallas_call(
        _onehot_block,
        grid=(grid,),
        in_specs=[pl.BlockSpec((ROWS, 1), lambda i: (i, 0))],
        out_specs=pl.BlockSpec((ROWS, NUM_CLASSES), lambda i: (i, 0)),
        out_shape=jax.ShapeDtypeStruct((BATCH, NUM_CLASSES), jnp.float32),
    )(x2)


# speedup vs baseline: 1.1070x; 1.1070x over previous
import jax
import jax.numpy as jnp
from jax import lax
from jax.experimental import pallas as pl
from jax.experimental.pallas import tpu as pltpu

NUM_CLASSES = 1000
PADDED = 1024
BATCH = 16384
ROWS = 512
NBLK = BATCH // ROWS
NBUF = 4


def _onehot_manual(x_ref, out_ref, xv_ref, scratch_ref, xsem, sems):
    pltpu.make_async_copy(x_ref, xv_ref, xsem).start()
    pltpu.make_async_copy(x_ref, xv_ref, xsem).wait()
    cols = lax.broadcasted_iota(jnp.int32, (ROWS, NUM_CLASSES), 1)
    for i in range(NBLK):
        buf = i % NBUF
        if i >= NBUF:
            pltpu.make_async_copy(
                scratch_ref.at[buf],
                out_ref.at[pl.ds((i - NBUF) * ROWS, ROWS), :],
                sems.at[buf],
            ).wait()
        x = xv_ref[pl.ds(i * ROWS, ROWS), :]
        scratch_ref[buf] = jnp.where(x == cols, 1.0, 0.0).astype(jnp.float32)
        pltpu.make_async_copy(
            scratch_ref.at[buf],
            out_ref.at[pl.ds(i * ROWS, ROWS), :],
            sems.at[buf],
        ).start()
    for i in range(NBLK - NBUF, NBLK):
        buf = i % NBUF
        pltpu.make_async_copy(
            scratch_ref.at[buf],
            out_ref.at[pl.ds(i * ROWS, ROWS), :],
            sems.at[buf],
        ).wait()


def kernel(x):
    x = x.astype(jnp.int32).reshape(BATCH, 1)
    return pl.pallas_call(
        _onehot_manual,
        in_specs=[pl.BlockSpec(memory_space=pl.ANY)],
        out_specs=pl.BlockSpec(memory_space=pl.ANY),
        out_shape=jax.ShapeDtypeStruct((BATCH, NUM_CLASSES), jnp.float32),
        scratch_shapes=[
            pltpu.VMEM((BATCH, 1), jnp.int32),
            pltpu.VMEM((NBUF, ROWS, NUM_CLASSES), jnp.float32),
            pltpu.SemaphoreType.DMA,
            pltpu.SemaphoreType.DMA((NBUF,)),
        ],
    )(x)


# speedup vs baseline: 1.2133x; 1.0961x over previous
import jax
import jax.numpy as jnp
from jax import lax
from jax.experimental import pallas as pl
from jax.experimental.pallas import tpu as pltpu

NUM_CLASSES = 1000
BATCH = 16384
ROWS = 128
NBLK = BATCH // ROWS
NBUF = 8


def _onehot_manual(x_ref, out_ref, xv_ref, xt_ref, scratch_ref, xsem, sems):
    pltpu.make_async_copy(x_ref, xv_ref, xsem).start()
    pltpu.make_async_copy(x_ref, xv_ref, xsem).wait()
    xt_ref[...] = xv_ref[...].T
    cols = lax.broadcasted_iota(jnp.int32, (ROWS, NUM_CLASSES), 1)
    for i in range(NBLK):
        buf = i % NBUF
        if i >= NBUF:
            pltpu.make_async_copy(
                scratch_ref.at[buf],
                out_ref.at[pl.ds((i - NBUF) * ROWS, ROWS), :],
                sems.at[buf],
            ).wait()
        x = xt_ref[:, i : i + 1]
        scratch_ref[buf] = jnp.where(x == cols, 1.0, 0.0).astype(jnp.float32)
        pltpu.make_async_copy(
            scratch_ref.at[buf],
            out_ref.at[pl.ds(i * ROWS, ROWS), :],
            sems.at[buf],
        ).start()
    for i in range(NBLK - NBUF, NBLK):
        buf = i % NBUF
        pltpu.make_async_copy(
            scratch_ref.at[buf],
            out_ref.at[pl.ds(i * ROWS, ROWS), :],
            sems.at[buf],
        ).wait()


def kernel(x):
    x = x.astype(jnp.int32).reshape(NBLK, ROWS)
    return pl.pallas_call(
        _onehot_manual,
        in_specs=[pl.BlockSpec(memory_space=pl.ANY)],
        out_specs=pl.BlockSpec(memory_space=pl.ANY),
        out_shape=jax.ShapeDtypeStruct((BATCH, NUM_CLASSES), jnp.float32),
        scratch_shapes=[
            pltpu.VMEM((NBLK, ROWS), jnp.int32),
            pltpu.VMEM((ROWS, NBLK), jnp.int32),
            pltpu.VMEM((NBUF, ROWS, NUM_CLASSES), jnp.float32),
            pltpu.SemaphoreType.DMA,
            pltpu.SemaphoreType.DMA((NBUF,)),
        ],
    )(x)


# speedup vs baseline: 4.6898x; 3.8652x over previous
import jax
import jax.numpy as jnp
from jax import lax
from jax.experimental import pallas as pl

NUM_CLASSES = 1000
BATCH = 16384
COLS = 1024


def _onehot_block(x_ref, out_ref):
    x = x_ref[...]
    rows = lax.broadcasted_iota(jnp.int32, (NUM_CLASSES, COLS), 0)
    out_ref[...] = jnp.where(x == rows, 1.0, 0.0).astype(jnp.float32)


def kernel(x):
    x = x.astype(jnp.int32).reshape(1, BATCH)
    grid = BATCH // COLS
    oh_t = pl.pallas_call(
        _onehot_block,
        grid=(grid,),
        in_specs=[pl.BlockSpec((1, COLS), lambda i: (0, i))],
        out_specs=pl.BlockSpec((NUM_CLASSES, COLS), lambda i: (0, i)),
        out_shape=jax.ShapeDtypeStruct((NUM_CLASSES, BATCH), jnp.float32),
    )(x)
    return oh_t.T
